# baseline (speedup 1.0000x reference)
import functools

import jax
import jax.numpy as jnp
from jax import lax
from jax.experimental import pallas as pl
from jax.experimental.pallas import tpu as pltpu
from jax.experimental.pallas import tpu_sc as plsc

N = 2048
E = 131072
D = 16
H = 8
NN = N * N
RB = 17
RROWS = 1 << RB
NPASS = (NN // RROWS) // 2
TPB = E // 16
DUMP0 = RROWS
ACC_ROWS = RROWS + 256
CBUF = TPB + 128

BE = 8192


BI = 16


def _tr_body(x_ref, o_ref):
    o_ref[...] = jnp.transpose(x_ref[...], (0, 1, 3, 2))


def _proj_body(ei_ref, x_ref, wt_ref, b_ref, proj_ref, flat_ref):
    proj_ref[...] = (
        jnp.dot(x_ref[...], wt_ref[...], preferred_element_type=jnp.float32)
        + b_ref[...]
    )
    flat_ref[...] = ei_ref[0, :] * N + ei_ref[1, :]


_sc_mesh = plsc.VectorSubcoreMesh(core_axis_name="c", subcore_axis_name="s")


@functools.partial(
    pl.kernel,
    out_type=jax.ShapeDtypeStruct((NN, H), jnp.float32),
    mesh=_sc_mesh,
    scratch_types=[
        pltpu.VMEM((TPB,), jnp.int32),
        pltpu.VMEM((CBUF,), jnp.int32),
        pltpu.VMEM((CBUF,), jnp.int32),
        pltpu.VMEM((128, H), jnp.float32),
        pltpu.VMEM((1024, H), jnp.float32),
        pltpu.VMEM_SHARED((ACC_ROWS, H), jnp.float32),
        pltpu.SemaphoreType.DMA,
    ],
    compiler_params=pltpu.CompilerParams(
        use_tc_tiling_on_sc=False, needs_layout_passes=False),
)
def _sc_scatter(flat_hbm, proj_hbm, zeros_hbm, out_hbm,
                idx_v, ceid, cidx, rowbuf, zero_v, acc, sem):
    c = lax.axis_index("c")
    s = lax.axis_index("s")
    base = s * TPB

    pltpu.sync_copy(flat_hbm.at[pl.ds(base, TPB)], idx_v)
    pltpu.sync_copy(zeros_hbm, zero_v)
    dump_vec = jnp.full((16,), DUMP0 + 8 * s, jnp.int32)
    zero_vec = jnp.zeros((16,), jnp.int32)
    lane = jnp.arange(16, dtype=jnp.int32)

    def init_body(v, carry):
        ceid[pl.ds(v * 16, 16)] = zero_vec
        return carry
    lax.fori_loop(0, CBUF // 16, init_body, 0)

    def pass_body(p, carry):
        r = 2 * p + c

        def zbody(q, carry2):
            pltpu.sync_copy(zero_v, acc.at[pl.ds(s * 8192 + q * 1024, 1024), :])
            return carry2
        lax.fori_loop(0, 8, zbody, 0)

        r_vec = jnp.full((16,), r, jnp.int32)

        def cbody(v, cursor):
            vec = plsc.load_gather(
                idx_v, [jnp.full((16,), v * 16, jnp.int32) + lane])
            rid = lax.shift_right_logical(vec, RB)
            m = rid == r_vec
            mi = m.astype(jnp.int32)
            pos = jnp.full((16,), cursor - 1, jnp.int32) + plsc.cumsum(mi)
            evec = jnp.full((16,), base + v * 16, jnp.int32) + lane
            plsc.store_scatter(ceid, [pos], evec, mask=m)
            plsc.store_scatter(
                cidx, [pos], jnp.bitwise_and(vec, RROWS - 1), mask=m)
            return cursor + jnp.sum(mi)
        cursor = lax.fori_loop(0, TPB // 16, cbody, 0)

        cur_vec = jnp.full((16,), cursor, jnp.int32) + lane
        plsc.store_scatter(ceid, [cur_vec], zero_vec)
        plsc.store_scatter(cidx, [cur_vec], dump_vec)

        plsc.subcore_barrier()

        n16 = (cursor + 15) // 16
        n128 = (cursor + 127) // 128

        def gbody(g, carry2):
            pltpu.async_copy(
                proj_hbm.at[ceid.at[pl.ds(g * 128, 128)]], rowbuf, sem).wait()
            tmax = jnp.minimum(8, n16 - g * 8)

            def tbody(t, carry3):
                idxv = plsc.load_gather(
                    cidx,
                    [jnp.full((16,), g * 128 + t * 16, jnp.int32) + lane])
                pltpu.sync_copy(
                    rowbuf.at[pl.ds(t * 16, 16), :], acc.at[idxv], add=True)
                return carry3
            return lax.fori_loop(0, tmax, tbody, carry2)
        lax.fori_loop(0, n128, gbody, 0)

        plsc.subcore_barrier()

        pltpu.sync_copy(
            acc.at[pl.ds(s * 8192, 8192), :],
            out_hbm.at[pl.ds(r * RROWS + s * 8192, 8192), :],
        )
        return carry
    lax.fori_loop(0, NPASS, pass_body, 0)


def kernel(edge_index, edge_attr, num_nodes, W, b):
    del num_nodes
    wt = W.T
    proj, flat = pl.pallas_call(
        _proj_body,
        grid=(E // BE,),
        in_specs=[
            pl.BlockSpec((2, BE), lambda g: (0, g)),
            pl.BlockSpec((BE, D), lambda g: (g, 0)),
            pl.BlockSpec((D, H), lambda g: (0, 0)),
            pl.BlockSpec((1, H), lambda g: (0, 0)),
        ],
        out_specs=[
            pl.BlockSpec((BE, H), lambda g: (g, 0)),
            pl.BlockSpec((BE,), lambda g: (g,)),
        ],
        out_shape=[
            jax.ShapeDtypeStruct((E, H), jnp.float32),
            jax.ShapeDtypeStruct((E,), jnp.int32),
        ],
    )(edge_index.astype(jnp.int32), edge_attr, wt, b.reshape(1, H))

    zeros_src = jnp.zeros((1024, H), jnp.float32)
    out2 = _sc_scatter(flat, proj, zeros_src)

    x4 = out2.reshape(N, N // 128, 128, H)
    out4 = pl.pallas_call(
        _tr_body,
        grid=(N // BI,),
        in_specs=[pl.BlockSpec((BI, N // 128, 128, H), lambda g: (g, 0, 0, 0))],
        out_specs=pl.BlockSpec((BI, N // 128, H, 128), lambda g: (g, 0, 0, 0)),
        out_shape=jax.ShapeDtypeStruct((N, N // 128, H, 128), jnp.float32),
    )(x4)
    return (
        out4.transpose(0, 2, 1, 3).reshape(N, H, N).transpose(0, 2, 1)
    )


# speedup vs baseline: 3.1360x; 3.1360x over previous
import functools

import jax
import jax.numpy as jnp
from jax import lax
from jax.experimental import pallas as pl
from jax.experimental.pallas import tpu as pltpu
from jax.experimental.pallas import tpu_sc as plsc

N = 2048
E = 131072
D = 16
H = 8
NN = N * N
RB = 17
RROWS = 1 << RB
NREG = NN // RROWS
NPASS = NREG // 2
TPB = E // 16
DUMP0 = RROWS
ACC_ROWS = RROWS + 256
SBUF = TPB + NREG * 128 + 16

BE = 8192


def _proj_body(ei_ref, x_ref, wt_ref, b_ref, proj_ref, flat_ref):
    proj_ref[...] = (
        jnp.dot(x_ref[...], wt_ref[...], preferred_element_type=jnp.float32)
        + b_ref[...]
    )
    flat_ref[...] = ei_ref[0, :] * N + ei_ref[1, :]


_sc_mesh = plsc.VectorSubcoreMesh(core_axis_name="c", subcore_axis_name="s")


@functools.partial(
    pl.kernel,
    out_type=jax.ShapeDtypeStruct((NN * H // 128, 128), jnp.float32),
    mesh=_sc_mesh,
    scratch_types=[
        pltpu.VMEM((TPB,), jnp.int32),
        pltpu.VMEM((SBUF,), jnp.int32),
        pltpu.VMEM((SBUF,), jnp.int32),
        pltpu.VMEM((NREG,), jnp.int32),
        pltpu.VMEM((NREG,), jnp.int32),
        pltpu.VMEM((NREG,), jnp.int32),
        pltpu.VMEM((NREG,), jnp.int32),
        pltpu.VMEM((128, H), jnp.float32),
        pltpu.VMEM((1024, H), jnp.float32),
        pltpu.VMEM((1024, H), jnp.float32),
        pltpu.VMEM((64, 128), jnp.float32),
        pltpu.VMEM_SHARED((ACC_ROWS, H), jnp.float32),
        pltpu.SemaphoreType.DMA,
    ],
    compiler_params=pltpu.CompilerParams(
        use_tc_tiling_on_sc=False, needs_layout_passes=False),
)
def _sc_scatter(flat_hbm, proj_hbm, zeros_hbm, out_hbm,
                idx_v, seid, sidx, cnt_v, off_v, pc_v, cur_v,
                rowbuf, zero_v, stage_v, obuf, acc, sem):
    c = lax.axis_index("c")
    s = lax.axis_index("s")
    base = s * TPB

    pltpu.sync_copy(flat_hbm.at[pl.ds(base, TPB)], idx_v)
    pltpu.sync_copy(zeros_hbm, zero_v)
    dump_vec = jnp.full((16,), DUMP0 + 8 * s, jnp.int32)
    zero_vec = jnp.zeros((16,), jnp.int32)
    ones_vec = jnp.full((16,), 1, jnp.int32)
    lane = jnp.arange(16, dtype=jnp.int32)

    def init_body(v, carry):
        seid[pl.ds(v * 16, 16)] = zero_vec
        sidx[pl.ds(v * 16, 16)] = dump_vec
        return carry
    lax.fori_loop(0, SBUF // 16, init_body, 0)
    cnt_v[pl.ds(0, 16)] = zero_vec
    cnt_v[pl.ds(16, 16)] = zero_vec

    def hist_body(v, carry):
        vec = plsc.load_gather(idx_v, [jnp.full((16,), v * 16, jnp.int32) + lane])
        rid = lax.shift_right_logical(vec, RB)
        plsc.addupdate_scatter(cnt_v, [rid], ones_vec)
        return carry
    lax.fori_loop(0, TPB // 16, hist_body, 0)

    c0 = cnt_v[pl.ds(0, 16)]
    c1 = cnt_v[pl.ds(16, 16)]
    p0 = lax.shift_left(lax.shift_right_logical(c0 + 127, 7), 7)
    p1 = lax.shift_left(lax.shift_right_logical(c1 + 127, 7), 7)
    o0 = plsc.cumsum(p0) - p0
    t0 = jnp.sum(p0)
    o1 = plsc.cumsum(p1) - p1 + jnp.full((16,), t0, jnp.int32)
    off_v[pl.ds(0, 16)] = o0
    off_v[pl.ds(16, 16)] = o1
    pc_v[pl.ds(0, 16)] = p0
    pc_v[pl.ds(16, 16)] = p1
    cur_v[pl.ds(0, 16)] = o0
    cur_v[pl.ds(16, 16)] = o1

    def sort_body(v, carry):
        vec = plsc.load_gather(idx_v, [jnp.full((16,), v * 16, jnp.int32) + lane])
        rid = lax.shift_right_logical(vec, RB)
        rank, _ = plsc.scan_count(rid)
        pos = plsc.load_gather(cur_v, [rid]) + rank - ones_vec
        geid = jnp.full((16,), base + v * 16, jnp.int32) + lane
        plsc.store_scatter(seid, [pos], geid)
        plsc.store_scatter(sidx, [pos], jnp.bitwise_and(vec, RROWS - 1))
        plsc.addupdate_scatter(cur_v, [rid], ones_vec)
        return carry
    lax.fori_loop(0, TPB // 16, sort_body, 0)

    def pass_body(p, carry):
        r = 2 * p + c

        def zbody(q, carry2):
            pltpu.sync_copy(zero_v, acc.at[pl.ds(s * 8192 + q * 1024, 1024), :])
            return carry2
        lax.fori_loop(0, 8, zbody, 0)

        r_splat = jnp.full((16,), r, jnp.int32)
        start = pl.multiple_of(
            lax.reduce_max(plsc.load_gather(off_v, [r_splat]), (0,)), 128)
        nch = lax.shift_right_logical(
            lax.reduce_max(plsc.load_gather(pc_v, [r_splat]), (0,)), 7)

        plsc.subcore_barrier()

        def gbody(g, carry2):
            o = pl.multiple_of(start + g * 128, 128)
            pltpu.async_copy(
                proj_hbm.at[seid.at[pl.ds(o, 128)]], rowbuf, sem).wait()

            def tbody(t, carry3):
                idxv = plsc.load_gather(
                    sidx, [jnp.full((16,), o + t * 16, jnp.int32) + lane])
                pltpu.sync_copy(
                    rowbuf.at[pl.ds(t * 16, 16), :], acc.at[idxv], add=True)
                return carry3
            return lax.fori_loop(0, 8, tbody, carry2)
        lax.fori_loop(0, nch, gbody, 0)

        plsc.subcore_barrier()

        def obody(ch, carry2):
            pltpu.sync_copy(acc.at[pl.ds(s * 8192 + ch * 1024, 1024), :],
                            stage_v)

            def wbody(w, carry3):
                jtl = w // 8
                h = w % 8
                hv = jnp.full((16,), h, jnp.int32)
                for q in range(8):
                    rows = jnp.full((16,), jtl * 128 + q * 16, jnp.int32) + lane
                    vec = plsc.load_gather(stage_v, [rows, hv])
                    obuf[w, pl.ds(q * 16, 16)] = vec
                return carry3
            lax.fori_loop(0, 64, wbody, 0)

            i = (r * RROWS + s * 8192 + ch * 1024) // N
            orow = i * 128 + (ch % 2) * 64
            pltpu.sync_copy(obuf, out_hbm.at[pl.ds(orow, 64), :])
            return carry2
        lax.fori_loop(0, 8, obody, 0)
        return carry
    lax.fori_loop(0, NPASS, pass_body, 0)


def kernel(edge_index, edge_attr, num_nodes, W, b):
    del num_nodes
    wt = W.T
    proj, flat = pl.pallas_call(
        _proj_body,
        grid=(E // BE,),
        in_specs=[
            pl.BlockSpec((2, BE), lambda g: (0, g)),
            pl.BlockSpec((BE, D), lambda g: (g, 0)),
            pl.BlockSpec((D, H), lambda g: (0, 0)),
            pl.BlockSpec((1, H), lambda g: (0, 0)),
        ],
        out_specs=[
            pl.BlockSpec((BE, H), lambda g: (g, 0)),
            pl.BlockSpec((BE,), lambda g: (g,)),
        ],
        out_shape=[
            jax.ShapeDtypeStruct((E, H), jnp.float32),
            jax.ShapeDtypeStruct((E,), jnp.int32),
        ],
    )(edge_index.astype(jnp.int32), edge_attr, wt, b.reshape(1, H))

    zeros_src = jnp.zeros((1024, H), jnp.float32)
    out128 = _sc_scatter(flat, proj, zeros_src)
    out4 = out128.reshape(N, N // 128, H, 128)
    return out4.transpose(0, 2, 1, 3).reshape(N, H, N).transpose(0, 2, 1)


# speedup vs baseline: 3.2195x; 1.0266x over previous
import functools

import jax
import jax.numpy as jnp
from jax import lax
from jax.experimental import pallas as pl
from jax.experimental.pallas import tpu as pltpu
from jax.experimental.pallas import tpu_sc as plsc

N = 2048
E = 131072
D = 16
H = 8
NN = N * N
RB = 17
RROWS = 1 << RB
NREG = NN // RROWS
NPASS = NREG // 2
TPB = E // 16
DUMP0 = RROWS
ACC_ROWS = RROWS + 256
SBUF = TPB + NREG * 128 + 16

BE = 8192


def _proj_body(ei_ref, x_ref, wt_ref, b_ref, proj_ref, flat_ref):
    proj_ref[...] = (
        jnp.dot(x_ref[...], wt_ref[...], preferred_element_type=jnp.float32)
        + b_ref[...]
    )
    flat_ref[...] = ei_ref[0, :] * N + ei_ref[1, :]


_sc_mesh = plsc.VectorSubcoreMesh(core_axis_name="c", subcore_axis_name="s")


@functools.partial(
    pl.kernel,
    out_type=jax.ShapeDtypeStruct((NN * H // 128, 128), jnp.float32),
    mesh=_sc_mesh,
    scratch_types=[
        pltpu.VMEM((TPB,), jnp.int32),
        pltpu.VMEM((SBUF,), jnp.int32),
        pltpu.VMEM((SBUF,), jnp.int32),
        pltpu.VMEM((NREG,), jnp.int32),
        pltpu.VMEM((NREG,), jnp.int32),
        pltpu.VMEM((NREG,), jnp.int32),
        pltpu.VMEM((NREG,), jnp.int32),
        pltpu.VMEM((128, H), jnp.float32),
        pltpu.VMEM((1024, H), jnp.float32),
        pltpu.VMEM((1024, H), jnp.float32),
        pltpu.VMEM((64, 128), jnp.float32),
        pltpu.VMEM_SHARED((ACC_ROWS, H), jnp.float32),
        pltpu.SemaphoreType.DMA,
        pltpu.SemaphoreType.DMA,
        pltpu.SemaphoreType.DMA,
    ],
    compiler_params=pltpu.CompilerParams(
        use_tc_tiling_on_sc=False, needs_layout_passes=False),
)
def _sc_scatter(flat_hbm, proj_hbm, zeros_hbm, out_hbm,
                idx_v, seid, sidx, cnt_v, off_v, pc_v, cur_v,
                rowbuf, zero_v, stage_v, obuf, acc, sem, semz, sems):
    c = lax.axis_index("c")
    s = lax.axis_index("s")
    base = s * TPB

    pltpu.sync_copy(flat_hbm.at[pl.ds(base, TPB)], idx_v)
    pltpu.sync_copy(zeros_hbm, zero_v)
    dump_vec = jnp.full((16,), DUMP0 + 8 * s, jnp.int32)
    zero_vec = jnp.zeros((16,), jnp.int32)
    ones_vec = jnp.full((16,), 1, jnp.int32)
    lane = jnp.arange(16, dtype=jnp.int32)

    def init_body(v, carry):
        seid[pl.ds(v * 16, 16)] = zero_vec
        sidx[pl.ds(v * 16, 16)] = dump_vec
        return carry
    lax.fori_loop(0, SBUF // 16, init_body, 0)
    cnt_v[pl.ds(0, 16)] = zero_vec
    cnt_v[pl.ds(16, 16)] = zero_vec

    def hist_body(v, carry):
        vec = plsc.load_gather(idx_v, [jnp.full((16,), v * 16, jnp.int32) + lane])
        rid = lax.shift_right_logical(vec, RB)
        plsc.addupdate_scatter(cnt_v, [rid], ones_vec)
        return carry
    lax.fori_loop(0, TPB // 16, hist_body, 0)

    c0 = cnt_v[pl.ds(0, 16)]
    c1 = cnt_v[pl.ds(16, 16)]
    p0 = lax.shift_left(lax.shift_right_logical(c0 + 127, 7), 7)
    p1 = lax.shift_left(lax.shift_right_logical(c1 + 127, 7), 7)
    o0 = plsc.cumsum(p0) - p0
    t0 = jnp.sum(p0)
    o1 = plsc.cumsum(p1) - p1 + jnp.full((16,), t0, jnp.int32)
    off_v[pl.ds(0, 16)] = o0
    off_v[pl.ds(16, 16)] = o1
    pc_v[pl.ds(0, 16)] = p0
    pc_v[pl.ds(16, 16)] = p1
    cur_v[pl.ds(0, 16)] = o0
    cur_v[pl.ds(16, 16)] = o1

    def sort_body(v, carry):
        vec = plsc.load_gather(idx_v, [jnp.full((16,), v * 16, jnp.int32) + lane])
        rid = lax.shift_right_logical(vec, RB)
        rank, _ = plsc.scan_count(rid)
        pos = plsc.load_gather(cur_v, [rid]) + rank - ones_vec
        geid = jnp.full((16,), base + v * 16, jnp.int32) + lane
        plsc.store_scatter(seid, [pos], geid)
        plsc.store_scatter(sidx, [pos], jnp.bitwise_and(vec, RROWS - 1))
        plsc.addupdate_scatter(cur_v, [rid], ones_vec)
        return carry
    lax.fori_loop(0, TPB // 16, sort_body, 0)

    def pass_body(p, carry):
        r = 2 * p + c

        def zbody(q, carry2):
            pltpu.async_copy(
                zero_v, acc.at[pl.ds(s * 8192 + q * 1024, 1024), :], semz)
            return carry2
        lax.fori_loop(0, 8, zbody, 0)

        def zdrain(q, carry2):
            pltpu.make_async_copy(
                zero_v, acc.at[pl.ds(s * 8192 + q * 1024, 1024), :], semz
            ).wait()
            return carry2
        lax.fori_loop(0, 8, zdrain, 0)

        r_splat = jnp.full((16,), r, jnp.int32)
        start = pl.multiple_of(
            lax.reduce_max(plsc.load_gather(off_v, [r_splat]), (0,)), 128)
        nch = lax.shift_right_logical(
            lax.reduce_max(plsc.load_gather(pc_v, [r_splat]), (0,)), 7)

        plsc.subcore_barrier()

        def gbody(g, carry2):
            o = pl.multiple_of(start + g * 128, 128)
            pltpu.async_copy(
                proj_hbm.at[seid.at[pl.ds(o, 128)]], rowbuf, sem).wait()

            def tbody(t, carry3):
                idxv = plsc.load_gather(
                    sidx, [jnp.full((16,), o + t * 16, jnp.int32) + lane])
                pltpu.async_copy(
                    rowbuf.at[pl.ds(t * 16, 16), :], acc.at[idxv], sems,
                    add=True)
                return carry3
            lax.fori_loop(0, 8, tbody, carry2)

            def tdrain(t, carry3):
                idxv = plsc.load_gather(
                    sidx, [jnp.full((16,), o + t * 16, jnp.int32) + lane])
                pltpu.make_async_copy(
                    rowbuf.at[pl.ds(t * 16, 16), :], acc.at[idxv], sems
                ).wait()
                return carry3
            return lax.fori_loop(0, 8, tdrain, carry2)
        lax.fori_loop(0, nch, gbody, 0)

        plsc.subcore_barrier()

        def obody(ch, carry2):
            pltpu.sync_copy(acc.at[pl.ds(s * 8192 + ch * 1024, 1024), :],
                            stage_v)

            def wbody(w, carry3):
                jtl = w // 8
                h = w % 8
                hv = jnp.full((16,), h, jnp.int32)
                for q in range(8):
                    rows = jnp.full((16,), jtl * 128 + q * 16, jnp.int32) + lane
                    vec = plsc.load_gather(stage_v, [rows, hv])
                    obuf[w, pl.ds(q * 16, 16)] = vec
                return carry3
            lax.fori_loop(0, 64, wbody, 0)

            i = (r * RROWS + s * 8192 + ch * 1024) // N
            orow = i * 128 + (ch % 2) * 64
            pltpu.sync_copy(obuf, out_hbm.at[pl.ds(orow, 64), :])
            return carry2
        lax.fori_loop(0, 8, obody, 0)
        return carry
    lax.fori_loop(0, NPASS, pass_body, 0)


def kernel(edge_index, edge_attr, num_nodes, W, b):
    del num_nodes
    wt = W.T
    proj, flat = pl.pallas_call(
        _proj_body,
        grid=(E // BE,),
        in_specs=[
            pl.BlockSpec((2, BE), lambda g: (0, g)),
            pl.BlockSpec((BE, D), lambda g: (g, 0)),
            pl.BlockSpec((D, H), lambda g: (0, 0)),
            pl.BlockSpec((1, H), lambda g: (0, 0)),
        ],
        out_specs=[
            pl.BlockSpec((BE, H), lambda g: (g, 0)),
            pl.BlockSpec((BE,), lambda g: (g,)),
        ],
        out_shape=[
            jax.ShapeDtypeStruct((E, H), jnp.float32),
            jax.ShapeDtypeStruct((E,), jnp.int32),
        ],
    )(edge_index.astype(jnp.int32), edge_attr, wt, b.reshape(1, H))

    zeros_src = jnp.zeros((1024, H), jnp.float32)
    out128 = _sc_scatter(flat, proj, zeros_src)
    out4 = out128.reshape(N, N // 128, H, 128)
    return out4.transpose(0, 2, 1, 3).reshape(N, H, N).transpose(0, 2, 1)


# speedup vs baseline: 3.2707x; 1.0159x over previous
import functools

import jax
import jax.numpy as jnp
from jax import lax
from jax.experimental import pallas as pl
from jax.experimental.pallas import tpu as pltpu
from jax.experimental.pallas import tpu_sc as plsc

N = 2048
E = 131072
D = 16
H = 8
NN = N * N
RB = 17
RROWS = 1 << RB
NREG = NN // RROWS
NPASS = NREG // 2
TPB = E // 16
DUMP0 = RROWS
ACC_ROWS = RROWS + 256
SBUF = TPB + NREG * 128 + 16

BE = 8192


def _proj_body(ei_ref, x_ref, wt_ref, b_ref, proj_ref, flat_ref):
    proj_ref[...] = (
        jnp.dot(x_ref[...], wt_ref[...], preferred_element_type=jnp.float32)
        + b_ref[...]
    )
    flat_ref[...] = ei_ref[0, :] * N + ei_ref[1, :]


_sc_mesh = plsc.VectorSubcoreMesh(core_axis_name="c", subcore_axis_name="s")


@functools.partial(
    pl.kernel,
    out_type=jax.ShapeDtypeStruct((NN * H // 128, 128), jnp.float32),
    mesh=_sc_mesh,
    scratch_types=[
        pltpu.VMEM((TPB,), jnp.int32),
        pltpu.VMEM((SBUF,), jnp.int32),
        pltpu.VMEM((SBUF,), jnp.int32),
        pltpu.VMEM((NREG,), jnp.int32),
        pltpu.VMEM((NREG,), jnp.int32),
        pltpu.VMEM((NREG,), jnp.int32),
        pltpu.VMEM((NREG,), jnp.int32),
        pltpu.VMEM((128, H), jnp.float32),
        pltpu.VMEM((1024, H), jnp.float32),
        pltpu.VMEM((1024, H), jnp.float32),
        pltpu.VMEM((64, 128), jnp.float32),
        pltpu.VMEM_SHARED((ACC_ROWS, H), jnp.float32),
        pltpu.SemaphoreType.DMA,
        pltpu.SemaphoreType.DMA,
        pltpu.SemaphoreType.DMA,
        pltpu.SemaphoreType.DMA,
    ],
    compiler_params=pltpu.CompilerParams(
        use_tc_tiling_on_sc=False, needs_layout_passes=False),
)
def _sc_scatter(flat_hbm, proj_hbm, zeros_hbm, out_hbm,
                idx_v, seid, sidx, cnt_v, off_v, pc_v, cur_v,
                rowbuf, zero_v, stage_v, obuf, acc, sem, semz, sems, semo):
    c = lax.axis_index("c")
    s = lax.axis_index("s")
    base = s * TPB

    pltpu.sync_copy(flat_hbm.at[pl.ds(base, TPB)], idx_v)
    pltpu.sync_copy(zeros_hbm, zero_v)
    dump_vec = jnp.full((16,), DUMP0 + 8 * s, jnp.int32)
    zero_vec = jnp.zeros((16,), jnp.int32)
    ones_vec = jnp.full((16,), 1, jnp.int32)
    lane = jnp.arange(16, dtype=jnp.int32)

    def init_body(v, carry):
        seid[pl.ds(v * 16, 16)] = zero_vec
        sidx[pl.ds(v * 16, 16)] = dump_vec
        return carry
    lax.fori_loop(0, SBUF // 16, init_body, 0)
    cnt_v[pl.ds(0, 16)] = zero_vec
    cnt_v[pl.ds(16, 16)] = zero_vec

    def hist_body(v, carry):
        vec = plsc.load_gather(idx_v, [jnp.full((16,), v * 16, jnp.int32) + lane])
        rid = lax.shift_right_logical(vec, RB)
        plsc.addupdate_scatter(cnt_v, [rid], ones_vec)
        return carry
    lax.fori_loop(0, TPB // 16, hist_body, 0)

    c0 = cnt_v[pl.ds(0, 16)]
    c1 = cnt_v[pl.ds(16, 16)]
    p0 = lax.shift_left(lax.shift_right_logical(c0 + 127, 7), 7)
    p1 = lax.shift_left(lax.shift_right_logical(c1 + 127, 7), 7)
    o0 = plsc.cumsum(p0) - p0
    t0 = jnp.sum(p0)
    o1 = plsc.cumsum(p1) - p1 + jnp.full((16,), t0, jnp.int32)
    off_v[pl.ds(0, 16)] = o0
    off_v[pl.ds(16, 16)] = o1
    pc_v[pl.ds(0, 16)] = p0
    pc_v[pl.ds(16, 16)] = p1
    cur_v[pl.ds(0, 16)] = o0
    cur_v[pl.ds(16, 16)] = o1

    def sort_body(v, carry):
        vec = plsc.load_gather(idx_v, [jnp.full((16,), v * 16, jnp.int32) + lane])
        rid = lax.shift_right_logical(vec, RB)
        rank, _ = plsc.scan_count(rid)
        pos = plsc.load_gather(cur_v, [rid]) + rank - ones_vec
        geid = jnp.full((16,), base + v * 16, jnp.int32) + lane
        plsc.store_scatter(seid, [pos], geid)
        plsc.store_scatter(sidx, [pos], jnp.bitwise_and(vec, RROWS - 1))
        plsc.addupdate_scatter(cur_v, [rid], ones_vec)
        return carry
    lax.fori_loop(0, TPB // 16, sort_body, 0)

    def pass_body(p, carry):
        r = 2 * p + c

        def zbody(q, carry2):
            pltpu.async_copy(
                zero_v, acc.at[pl.ds(s * 8192 + q * 1024, 1024), :], semz)
            return carry2
        lax.fori_loop(0, 8, zbody, 0)

        def zdrain(q, carry2):
            pltpu.make_async_copy(
                zero_v, acc.at[pl.ds(s * 8192 + q * 1024, 1024), :], semz
            ).wait()
            return carry2
        lax.fori_loop(0, 8, zdrain, 0)

        r_splat = jnp.full((16,), r, jnp.int32)
        start = pl.multiple_of(
            lax.reduce_max(plsc.load_gather(off_v, [r_splat]), (0,)), 128)
        nch = lax.shift_right_logical(
            lax.reduce_max(plsc.load_gather(pc_v, [r_splat]), (0,)), 7)

        plsc.subcore_barrier()

        def gbody(g, carry2):
            o = pl.multiple_of(start + g * 128, 128)
            pltpu.async_copy(
                proj_hbm.at[seid.at[pl.ds(o, 128)]], rowbuf, sem).wait()

            def tbody(t, carry3):
                idxv = plsc.load_gather(
                    sidx, [jnp.full((16,), o + t * 16, jnp.int32) + lane])
                pltpu.async_copy(
                    rowbuf.at[pl.ds(t * 16, 16), :], acc.at[idxv], sems,
                    add=True)
                return carry3
            lax.fori_loop(0, 8, tbody, carry2)

            def tdrain(t, carry3):
                idxv = plsc.load_gather(
                    sidx, [jnp.full((16,), o + t * 16, jnp.int32) + lane])
                pltpu.make_async_copy(
                    rowbuf.at[pl.ds(t * 16, 16), :], acc.at[idxv], sems
                ).wait()
                return carry3
            return lax.fori_loop(0, 8, tdrain, carry2)
        lax.fori_loop(0, nch, gbody, 0)

        plsc.subcore_barrier()

        def obody(ch, carry2):
            pltpu.sync_copy(acc.at[pl.ds(s * 8192 + ch * 1024, 1024), :],
                            stage_v)
            i = (r * RROWS + s * 8192 + ch * 1024) // N
            orow = i * 128 + (ch % 2) * 64

            for half in range(2):
                def wbody(w, carry3, half=half):
                    wg = half * 32 + w
                    jtl = wg // 8
                    h = wg % 8
                    hv = jnp.full((16,), h, jnp.int32)
                    for q in range(8):
                        rows = (
                            jnp.full((16,), jtl * 128 + q * 16, jnp.int32)
                            + lane
                        )
                        vec = plsc.load_gather(stage_v, [rows, hv])
                        obuf[wg, pl.ds(q * 16, 16)] = vec
                    return carry3
                lax.fori_loop(0, 32, wbody, 0)
                pltpu.async_copy(
                    obuf.at[pl.ds(half * 32, 32), :],
                    out_hbm.at[pl.ds(orow + half * 32, 32), :], semo)
            for half in range(2):
                pltpu.make_async_copy(
                    obuf.at[pl.ds(half * 32, 32), :],
                    out_hbm.at[pl.ds(orow + half * 32, 32), :], semo,
                ).wait()
            return carry2
        lax.fori_loop(0, 8, obody, 0)
        return carry
    lax.fori_loop(0, NPASS, pass_body, 0)


def kernel(edge_index, edge_attr, num_nodes, W, b):
    del num_nodes
    wt = W.T
    proj, flat = pl.pallas_call(
        _proj_body,
        grid=(E // BE,),
        in_specs=[
            pl.BlockSpec((2, BE), lambda g: (0, g)),
            pl.BlockSpec((BE, D), lambda g: (g, 0)),
            pl.BlockSpec((D, H), lambda g: (0, 0)),
            pl.BlockSpec((1, H), lambda g: (0, 0)),
        ],
        out_specs=[
            pl.BlockSpec((BE, H), lambda g: (g, 0)),
            pl.BlockSpec((BE,), lambda g: (g,)),
        ],
        out_shape=[
            jax.ShapeDtypeStruct((E, H), jnp.float32),
            jax.ShapeDtypeStruct((E,), jnp.int32),
        ],
    )(edge_index.astype(jnp.int32), edge_attr, wt, b.reshape(1, H))

    zeros_src = jnp.zeros((1024, H), jnp.float32)
    out128 = _sc_scatter(flat, proj, zeros_src)
    out4 = out128.reshape(N, N // 128, H, 128)
    return out4.transpose(0, 2, 1, 3).reshape(N, H, N).transpose(0, 2, 1)


# speedup vs baseline: 3.5141x; 1.0744x over previous
import functools

import jax
import jax.numpy as jnp
from jax import lax
from jax.experimental import pallas as pl
from jax.experimental.pallas import tpu as pltpu
from jax.experimental.pallas import tpu_sc as plsc

N = 2048
E = 131072
D = 16
H = 8
NN = N * N
RB = 17
RROWS = 1 << RB
NREG = NN // RROWS
NPASS = NREG // 2
TPB = E // 16
DUMP0 = RROWS
ACC_ROWS = RROWS + 256
SBUF = TPB + NREG * 128 + 16

BE = 8192


def _proj_body(ei_ref, xt_ref, wt_ref, b_ref, proj_ref, flat_ref):
    proj_ref[...] = (
        lax.dot_general(
            xt_ref[...], wt_ref[...], (((0,), (0,)), ((), ())),
            preferred_element_type=jnp.float32,
        )
        + b_ref[...]
    )
    flat_ref[...] = ei_ref[0, :] * N + ei_ref[1, :]


_sc_mesh = plsc.VectorSubcoreMesh(core_axis_name="c", subcore_axis_name="s")


@functools.partial(
    pl.kernel,
    out_type=jax.ShapeDtypeStruct((NN * H // 128, 128), jnp.float32),
    mesh=_sc_mesh,
    scratch_types=[
        pltpu.VMEM((TPB,), jnp.int32),
        pltpu.VMEM((SBUF,), jnp.int32),
        pltpu.VMEM((SBUF,), jnp.int32),
        pltpu.VMEM((NREG,), jnp.int32),
        pltpu.VMEM((NREG,), jnp.int32),
        pltpu.VMEM((NREG,), jnp.int32),
        pltpu.VMEM((NREG,), jnp.int32),
        pltpu.VMEM((128, H), jnp.float32),
        pltpu.VMEM((1024, H), jnp.float32),
        pltpu.VMEM((1024, H), jnp.float32),
        pltpu.VMEM((64, 128), jnp.float32),
        pltpu.VMEM_SHARED((ACC_ROWS, H), jnp.float32),
        pltpu.SemaphoreType.DMA,
        pltpu.SemaphoreType.DMA,
        pltpu.SemaphoreType.DMA,
        pltpu.SemaphoreType.DMA,
    ],
    compiler_params=pltpu.CompilerParams(
        use_tc_tiling_on_sc=False, needs_layout_passes=False),
)
def _sc_scatter(flat_hbm, proj_hbm, zeros_hbm, out_hbm,
                idx_v, seid, sidx, cnt_v, off_v, pc_v, cur_v,
                rowbuf, zero_v, stage_v, obuf, acc, sem, semz, sems, semo):
    c = lax.axis_index("c")
    s = lax.axis_index("s")
    base = s * TPB

    pltpu.sync_copy(flat_hbm.at[pl.ds(base, TPB)], idx_v)
    pltpu.sync_copy(zeros_hbm, zero_v)
    dump_vec = jnp.full((16,), DUMP0 + 8 * s, jnp.int32)
    zero_vec = jnp.zeros((16,), jnp.int32)
    ones_vec = jnp.full((16,), 1, jnp.int32)
    lane = jnp.arange(16, dtype=jnp.int32)

    def init_body(v, carry):
        seid[pl.ds(v * 16, 16)] = zero_vec
        sidx[pl.ds(v * 16, 16)] = dump_vec
        return carry
    lax.fori_loop(0, SBUF // 16, init_body, 0)
    cnt_v[pl.ds(0, 16)] = zero_vec
    cnt_v[pl.ds(16, 16)] = zero_vec

    def hist_body(v, carry):
        vec = plsc.load_gather(idx_v, [jnp.full((16,), v * 16, jnp.int32) + lane])
        rid = lax.shift_right_logical(vec, RB)
        plsc.addupdate_scatter(cnt_v, [rid], ones_vec)
        return carry
    lax.fori_loop(0, TPB // 16, hist_body, 0)

    c0 = cnt_v[pl.ds(0, 16)]
    c1 = cnt_v[pl.ds(16, 16)]
    p0 = lax.shift_left(lax.shift_right_logical(c0 + 127, 7), 7)
    p1 = lax.shift_left(lax.shift_right_logical(c1 + 127, 7), 7)
    o0 = plsc.cumsum(p0) - p0
    t0 = jnp.sum(p0)
    o1 = plsc.cumsum(p1) - p1 + jnp.full((16,), t0, jnp.int32)
    off_v[pl.ds(0, 16)] = o0
    off_v[pl.ds(16, 16)] = o1
    pc_v[pl.ds(0, 16)] = p0
    pc_v[pl.ds(16, 16)] = p1
    cur_v[pl.ds(0, 16)] = o0
    cur_v[pl.ds(16, 16)] = o1

    def sort_body(v, carry):
        vec = plsc.load_gather(idx_v, [jnp.full((16,), v * 16, jnp.int32) + lane])
        rid = lax.shift_right_logical(vec, RB)
        rank, _ = plsc.scan_count(rid)
        pos = plsc.load_gather(cur_v, [rid]) + rank - ones_vec
        geid = jnp.full((16,), base + v * 16, jnp.int32) + lane
        plsc.store_scatter(seid, [pos], geid)
        plsc.store_scatter(sidx, [pos], jnp.bitwise_and(vec, RROWS - 1))
        plsc.addupdate_scatter(cur_v, [rid], ones_vec)
        return carry
    lax.fori_loop(0, TPB // 16, sort_body, 0)

    def pass_body(p, carry):
        r = 2 * p + c

        def zbody(q, carry2):
            pltpu.async_copy(
                zero_v, acc.at[pl.ds(s * 8192 + q * 1024, 1024), :], semz)
            return carry2
        lax.fori_loop(0, 8, zbody, 0)

        def zdrain(q, carry2):
            pltpu.make_async_copy(
                zero_v, acc.at[pl.ds(s * 8192 + q * 1024, 1024), :], semz
            ).wait()
            return carry2
        lax.fori_loop(0, 8, zdrain, 0)

        r_splat = jnp.full((16,), r, jnp.int32)
        start = pl.multiple_of(
            lax.reduce_max(plsc.load_gather(off_v, [r_splat]), (0,)), 128)
        nch = lax.shift_right_logical(
            lax.reduce_max(plsc.load_gather(pc_v, [r_splat]), (0,)), 7)

        plsc.subcore_barrier()

        def gbody(g, carry2):
            o = pl.multiple_of(start + g * 128, 128)
            pltpu.async_copy(
                proj_hbm.at[seid.at[pl.ds(o, 128)]], rowbuf, sem).wait()

            def tbody(t, carry3):
                idxv = plsc.load_gather(
                    sidx, [jnp.full((16,), o + t * 16, jnp.int32) + lane])
                pltpu.async_copy(
                    rowbuf.at[pl.ds(t * 16, 16), :], acc.at[idxv], sems,
                    add=True)
                return carry3
            lax.fori_loop(0, 8, tbody, carry2)

            def tdrain(t, carry3):
                idxv = plsc.load_gather(
                    sidx, [jnp.full((16,), o + t * 16, jnp.int32) + lane])
                pltpu.make_async_copy(
                    rowbuf.at[pl.ds(t * 16, 16), :], acc.at[idxv], sems
                ).wait()
                return carry3
            return lax.fori_loop(0, 8, tdrain, carry2)
        lax.fori_loop(0, nch, gbody, 0)

        plsc.subcore_barrier()

        def obody(ch, carry2):
            pltpu.sync_copy(acc.at[pl.ds(s * 8192 + ch * 1024, 1024), :],
                            stage_v)
            i = (r * RROWS + s * 8192 + ch * 1024) // N
            orow = i * 128 + (ch % 2) * 64

            for half in range(2):
                def wbody(w, carry3, half=half):
                    wg = half * 32 + w
                    jtl = wg // 8
                    h = wg % 8
                    hv = jnp.full((16,), h, jnp.int32)
                    for q in range(8):
                        rows = (
                            jnp.full((16,), jtl * 128 + q * 16, jnp.int32)
                            + lane
                        )
                        vec = plsc.load_gather(stage_v, [rows, hv])
                        obuf[wg, pl.ds(q * 16, 16)] = vec
                    return carry3
                lax.fori_loop(0, 32, wbody, 0)
                pltpu.async_copy(
                    obuf.at[pl.ds(half * 32, 32), :],
                    out_hbm.at[pl.ds(orow + half * 32, 32), :], semo)
            for half in range(2):
                pltpu.make_async_copy(
                    obuf.at[pl.ds(half * 32, 32), :],
                    out_hbm.at[pl.ds(orow + half * 32, 32), :], semo,
                ).wait()
            return carry2
        lax.fori_loop(0, 8, obody, 0)
        return carry
    lax.fori_loop(0, NPASS, pass_body, 0)


def kernel(edge_index, edge_attr, num_nodes, W, b):
    del num_nodes
    wt = W.T
    proj, flat = pl.pallas_call(
        _proj_body,
        grid=(E // BE,),
        in_specs=[
            pl.BlockSpec((2, BE), lambda g: (0, g)),
            pl.BlockSpec((D, BE), lambda g: (0, g)),
            pl.BlockSpec((D, H), lambda g: (0, 0)),
            pl.BlockSpec((1, H), lambda g: (0, 0)),
        ],
        out_specs=[
            pl.BlockSpec((BE, H), lambda g: (g, 0)),
            pl.BlockSpec((BE,), lambda g: (g,)),
        ],
        out_shape=[
            jax.ShapeDtypeStruct((E, H), jnp.float32),
            jax.ShapeDtypeStruct((E,), jnp.int32),
        ],
    )(edge_index.astype(jnp.int32), edge_attr.T, wt, b.reshape(1, H))

    zeros_src = jnp.zeros((1024, H), jnp.float32)
    out128 = _sc_scatter(flat, proj, zeros_src)
    out4 = out128.reshape(N, N // 128, H, 128)
    return out4.transpose(0, 2, 1, 3).reshape(N, H, N).transpose(0, 2, 1)
